# baseline (speedup 1.0000x reference)
import functools

import jax
import jax.numpy as jnp
from jax import lax
from jax.experimental import pallas as pl
from jax.experimental.pallas import tpu as pltpu
from jax.experimental.pallas import tpu_sc as plsc

N = 2048
D = 32
K = 16
BR = 512

NC = 2
NS = 16
L = 16
NW = NC * NS
RPW = N // NW
CH = RPW // 2

NEG_INF = float("-inf")


def _topk_body(x_blk, x_full, out_idx):
    r = pl.program_id(0)
    xb = x_blk[...]
    xf = x_full[...]
    g = jax.lax.dot_general(
        xb, xf, (((1,), (1,)), ((), ())),
        precision=lax.Precision.HIGHEST,
        preferred_element_type=jnp.float32)
    nfh = 0.5 * jnp.sum(xf * xf, axis=1)[None, :]
    s = g - nfh
    cols = lax.broadcasted_iota(jnp.int32, (BR, N), 1)
    rows = r * BR + lax.broadcasted_iota(jnp.int32, (BR, 1), 0)
    s = jnp.where(cols == rows, NEG_INF, s)
    lane16 = lax.broadcasted_iota(jnp.int32, (BR, K), 1)
    acc = jnp.zeros((BR, K), dtype=jnp.int32)
    for t in range(K):
        picked = jnp.argmax(s, axis=1).astype(jnp.int32)[:, None]
        acc = jnp.where(lane16 == t, picked, acc)
        s = jnp.where(cols == picked, NEG_INF, s)
    out_idx[...] = acc


def _norm_body(a_blk, out_ahat):
    a = a_blk[...]
    rowsum = jnp.sum(a, axis=1, keepdims=True)
    out_ahat[...] = a * (1.0 / rowsum)


def _adj_sc_body(tk_hbm, a_hbm, tk_v, abuf):
    wid = lax.axis_index("s") * NC + lax.axis_index("c")
    pltpu.sync_copy(tk_hbm, tk_v)
    ones = jnp.ones((L,), jnp.float32)
    zeros = jnp.zeros((L,), jnp.float32)
    for chunk in range(2):
        r0 = wid * RPW + chunk * CH

        @plsc.parallel_loop(0, CH)
        def _(ri):
            for j in range(N // L):
                abuf[ri, pl.ds(j * L, L)] = zeros

        @plsc.parallel_loop(0, CH)
        def _(ri):
            v = tk_v[r0 + ri, :]
            plsc.store_scatter(abuf, [jnp.full((L,), ri, jnp.int32), v], ones)

        @plsc.parallel_loop(0, N, unroll=16)
        def _(i):
            v = tk_v[i, :]
            m = (v >= r0) & (v < r0 + CH)
            rows = jnp.where(m, v - r0, 0)
            cols = jnp.full((L,), i, jnp.int32)
            plsc.store_scatter(abuf, [rows, cols], ones, mask=m)

        pltpu.sync_copy(abuf, a_hbm.at[pl.ds(r0, CH), :])


@jax.jit
def kernel(x):
    topk = pl.pallas_call(
        _topk_body,
        grid=(N // BR,),
        in_specs=[
            pl.BlockSpec((BR, D), lambda r: (r, 0)),
            pl.BlockSpec((N, D), lambda r: (0, 0)),
        ],
        out_specs=pl.BlockSpec((BR, K), lambda r: (r, 0)),
        out_shape=jax.ShapeDtypeStruct((N, K), jnp.int32),
    )(x, x)
    adj_call = pl.kernel(
        _adj_sc_body,
        out_type=jax.ShapeDtypeStruct((N, N), jnp.float32),
        mesh=plsc.VectorSubcoreMesh(core_axis_name="c", subcore_axis_name="s"),
        compiler_params=pltpu.CompilerParams(
            use_tc_tiling_on_sc=False, needs_layout_passes=False),
        scratch_types=[
            pltpu.VMEM((N, K), jnp.int32),
            pltpu.VMEM((CH, N), jnp.float32),
        ],
    )
    a = adj_call(topk)
    ahat = pl.pallas_call(
        _norm_body,
        grid=(N // BR,),
        in_specs=[pl.BlockSpec((BR, N), lambda r: (r, 0))],
        out_specs=pl.BlockSpec((BR, N), lambda r: (r, 0)),
        out_shape=jax.ShapeDtypeStruct((N, N), jnp.float32),
    )(a)
    return (a, ahat)


# speedup vs baseline: 1.0159x; 1.0159x over previous
import functools

import jax
import jax.numpy as jnp
from jax import lax
from jax.experimental import pallas as pl
from jax.experimental.pallas import tpu as pltpu
from jax.experimental.pallas import tpu_sc as plsc

N = 2048
D = 32
K = 16
BR = 512

NC = 2
NS = 16
L = 16
NW = NC * NS
RPW = N // NW
CH = RPW // 2

NEG_INF = float("-inf")


def _topk_body(x_blk, x_full, out_idx):
    r = pl.program_id(0)
    xb = x_blk[...]
    xf = x_full[...]
    g = jax.lax.dot_general(
        xb, xf, (((1,), (1,)), ((), ())),
        precision=lax.Precision.HIGHEST,
        preferred_element_type=jnp.float32)
    nf = jnp.sum(xf * xf, axis=1)[None, :]
    s = 2.0 * g - nf
    cols = lax.broadcasted_iota(jnp.int32, (BR, N), 1)
    rows = r * BR + lax.broadcasted_iota(jnp.int32, (BR, 1), 0)
    s = jnp.where(cols == rows, NEG_INF, s)
    lane16 = lax.broadcasted_iota(jnp.int32, (BR, K), 1)
    acc = jnp.zeros((BR, K), dtype=jnp.int32)
    for t in range(K):
        picked = jnp.argmax(s, axis=1).astype(jnp.int32)[:, None]
        acc = jnp.where(lane16 == t, picked, acc)
        s = jnp.where(cols == picked, NEG_INF, s)
    out_idx[...] = acc


def _norm_body(a_blk, out_ahat):
    a = a_blk[...]
    rowsum = jnp.sum(a, axis=1, keepdims=True)
    out_ahat[...] = a * (1.0 / rowsum)


def _adj_sc_body(tk_hbm, a_hbm, tk_v, abuf):
    wid = lax.axis_index("s") * NC + lax.axis_index("c")
    pltpu.sync_copy(tk_hbm, tk_v)
    ones = jnp.ones((L,), jnp.float32)
    zeros = jnp.zeros((L,), jnp.float32)
    for chunk in range(2):
        r0 = wid * RPW + chunk * CH

        @plsc.parallel_loop(0, CH)
        def _(ri):
            for j in range(N // L):
                abuf[ri, pl.ds(j * L, L)] = zeros

        @plsc.parallel_loop(0, CH)
        def _(ri):
            v = tk_v[r0 + ri, :]
            plsc.store_scatter(abuf, [jnp.full((L,), ri, jnp.int32), v], ones)

        @plsc.parallel_loop(0, N, unroll=8)
        def _(i):
            v = tk_v[i, :]
            m = (v >= r0) & (v < r0 + CH)
            rows = jnp.where(m, v - r0, 0)
            cols = jnp.full((L,), i, jnp.int32)
            plsc.store_scatter(abuf, [rows, cols], ones, mask=m)

        pltpu.sync_copy(abuf, a_hbm.at[pl.ds(r0, CH), :])


@jax.jit
def kernel(x):
    topk = pl.pallas_call(
        _topk_body,
        grid=(N // BR,),
        in_specs=[
            pl.BlockSpec((BR, D), lambda r: (r, 0)),
            pl.BlockSpec((N, D), lambda r: (0, 0)),
        ],
        out_specs=pl.BlockSpec((BR, K), lambda r: (r, 0)),
        out_shape=jax.ShapeDtypeStruct((N, K), jnp.int32),
    )(x, x)
    adj_call = pl.kernel(
        _adj_sc_body,
        out_type=jax.ShapeDtypeStruct((N, N), jnp.float32),
        mesh=plsc.VectorSubcoreMesh(core_axis_name="c", subcore_axis_name="s"),
        compiler_params=pltpu.CompilerParams(
            use_tc_tiling_on_sc=False, needs_layout_passes=False),
        scratch_types=[
            pltpu.VMEM((N, K), jnp.int32),
            pltpu.VMEM((CH, N), jnp.float32),
        ],
    )
    a = adj_call(topk)
    ahat = pl.pallas_call(
        _norm_body,
        grid=(N // BR,),
        in_specs=[pl.BlockSpec((BR, N), lambda r: (r, 0))],
        out_specs=pl.BlockSpec((BR, N), lambda r: (r, 0)),
        out_shape=jax.ShapeDtypeStruct((N, N), jnp.float32),
    )(a)
    return (a, ahat)
